# baseline (speedup 1.0000x reference)
import functools

import jax
import jax.numpy as jnp
from jax import lax
from jax.experimental import pallas as pl
from jax.experimental.pallas import tpu as pltpu
from jax.experimental.pallas import tpu_sc as plsc

N_EXPERTS = 64
N_GROUPS = 8
GROUP_SIZE = N_EXPERTS // N_GROUPS
N_LIMITED_GROUPS = 4
TOPK = 8
BLK = 2048
N_TOK = 32768
NC = 2
NS = 16
NW = NC * NS
TPW = N_TOK // NW
L = 16


def _logits_body(x_ref, w_ref, lt_ref):
    x = x_ref[...]
    w = w_ref[...]
    lt_ref[...] = jax.lax.dot_general(
        w, x, (((1,), (1,)), ((), ())), preferred_element_type=jnp.float32)


@functools.lru_cache(maxsize=1)
def _make_sc_route():
    mesh = plsc.VectorSubcoreMesh(core_axis_name="c", subcore_axis_name="s")

    @functools.partial(
        pl.kernel,
        mesh=mesh,
        out_type=(
            jax.ShapeDtypeStruct((TOPK, N_TOK), jnp.float32),
            jax.ShapeDtypeStruct((TOPK, N_TOK), jnp.int32),
        ),
        scratch_types=[
            pltpu.VMEM((N_EXPERTS, TPW), jnp.float32),
            pltpu.VMEM((TOPK, TPW), jnp.float32),
            pltpu.VMEM((TOPK, TPW), jnp.int32),
        ],
    )
    def sc_route(lt_hbm, wt_hbm, it_hbm, lbuf, wbuf, ibuf):
        wid = lax.axis_index("s") * NC + lax.axis_index("c")
        base = wid * TPW
        pltpu.sync_copy(lt_hbm.at[:, pl.ds(base, TPW)], lbuf)

        neg = jnp.full((L,), -jnp.inf, jnp.float32)
        one = jnp.full((L,), 1.0, jnp.float32)
        lim = jnp.full((L,), N_LIMITED_GROUPS, jnp.int32)
        zero_i = jnp.zeros((L,), jnp.int32)

        def body(j, carry):
            off = j * L
            s = [lbuf[e, pl.ds(off, L)] for e in range(N_EXPERTS)]
            gm = []
            for g in range(N_GROUPS):
                v = s[g * GROUP_SIZE]
                for t in range(1, GROUP_SIZE):
                    v = jnp.maximum(v, s[g * GROUP_SIZE + t])
                gm.append(v)
            one_i = jnp.full((L,), 1, jnp.int32)
            masks = []
            for g in range(N_GROUPS):
                rank = zero_i
                for h in range(N_GROUPS):
                    if h == g:
                        continue
                    beats = (gm[h] >= gm[g]) if h < g else (gm[h] > gm[g])
                    rank = rank + jnp.where(beats, one_i, zero_i)
                masks.append(rank < lim)
            ms = [jnp.where(masks[e // GROUP_SIZE], s[e], neg)
                  for e in range(N_EXPERTS)]
            wk, ik = [], []
            for _ in range(TOPK):
                m = ms[0]
                for e in range(1, N_EXPERTS):
                    m = jnp.maximum(m, ms[e])
                idx = jnp.full((L,), N_EXPERTS, jnp.int32)
                for e in range(N_EXPERTS - 1, -1, -1):
                    idx = jnp.where(ms[e] == m,
                                    jnp.full((L,), e, jnp.int32), idx)
                for e in range(N_EXPERTS):
                    ms[e] = jnp.where(
                        idx == jnp.full((L,), e, jnp.int32), neg, ms[e])
                wk.append(one / (one + jnp.exp(jnp.zeros((L,), jnp.float32) - m)))
                ik.append(idx)
            tot = wk[0]
            for k in range(1, TOPK):
                tot = tot + wk[k]
            for k in range(TOPK):
                wbuf[k, pl.ds(off, L)] = wk[k] / tot
                ibuf[k, pl.ds(off, L)] = ik[k]
            return carry

        lax.fori_loop(0, TPW // L, body, 0)

        pltpu.sync_copy(wbuf, wt_hbm.at[:, pl.ds(base, TPW)])
        pltpu.sync_copy(ibuf, it_hbm.at[:, pl.ds(base, TPW)])

    return sc_route


@jax.jit
def kernel(x, W):
    n_tok, d = x.shape
    lt = pl.pallas_call(
        _logits_body,
        grid=(n_tok // BLK,),
        in_specs=[
            pl.BlockSpec((BLK, d), lambda i: (i, 0)),
            pl.BlockSpec((N_EXPERTS, d), lambda i: (0, 0)),
        ],
        out_specs=pl.BlockSpec((N_EXPERTS, BLK), lambda i: (0, i)),
        out_shape=jax.ShapeDtypeStruct((N_EXPERTS, n_tok), jnp.float32),
    )(x, W)
    wt, it = _make_sc_route()(lt)
    return wt.T.astype(x.dtype), it.T


# speedup vs baseline: 1.8141x; 1.8141x over previous
import jax
import jax.numpy as jnp
from jax.experimental import pallas as pl
from jax.experimental.pallas import tpu as pltpu

N_EXPERTS = 64
N_GROUPS = 8
GROUP_SIZE = N_EXPERTS // N_GROUPS
N_LIMITED_GROUPS = 4
TOPK = 8
BLK = 2048


def _gate_body(x_ref, w_ref, wt_ref, it_ref):
    x = x_ref[...]
    w = w_ref[...]
    logits = jax.lax.dot_general(
        w, x, (((1,), (1,)), ((), ())), preferred_element_type=jnp.float32)
    s = 1.0 / (1.0 + jnp.exp(-logits))

    s3 = s.reshape(N_GROUPS, GROUP_SIZE, BLK)
    gs = jnp.max(s3, axis=1)

    gh = gs[:, None, :]
    gg = gs[None, :, :]
    hi = jax.lax.broadcasted_iota(jnp.int32, (N_GROUPS, N_GROUPS, 1), 0)
    gi = jax.lax.broadcasted_iota(jnp.int32, (N_GROUPS, N_GROUPS, 1), 1)
    beats = (gh > gg) | ((gh == gg) & (hi < gi))
    grank = jnp.sum(beats.astype(jnp.int32), axis=0)
    gmask = grank < N_LIMITED_GROUPS
    m64 = jnp.broadcast_to(
        gmask[:, None, :], (N_GROUPS, GROUP_SIZE, BLK)).reshape(N_EXPERTS, BLK)
    ms = jnp.where(m64, s, 0.0)

    eidx = jax.lax.broadcasted_iota(jnp.int32, (N_EXPERTS, BLK), 0)
    ws, ids = [], []
    for _ in range(TOPK):
        m = jnp.max(ms, axis=0)
        idx = jnp.min(jnp.where(ms == m[None, :], eidx, N_EXPERTS), axis=0)
        sel = eidx == idx[None, :]
        wk = jnp.max(jnp.where(sel, s, -1.0), axis=0)
        ms = jnp.where(sel, -1.0, ms)
        ws.append(wk)
        ids.append(idx)
    wstack = jnp.stack(ws, axis=0)
    istack = jnp.stack(ids, axis=0)
    total = jnp.sum(wstack, axis=0, keepdims=True)
    wt_ref[...] = wstack / total
    it_ref[...] = istack


@jax.jit
def kernel(x, W):
    n_tok, d = x.shape
    grid = (n_tok // BLK,)
    wt, it = pl.pallas_call(
        _gate_body,
        grid=grid,
        compiler_params=pltpu.CompilerParams(
            dimension_semantics=("parallel",)),
        in_specs=[
            pl.BlockSpec((BLK, d), lambda i: (i, 0)),
            pl.BlockSpec((N_EXPERTS, d), lambda i: (0, 0)),
        ],
        out_specs=[
            pl.BlockSpec((TOPK, BLK), lambda i: (0, i)),
            pl.BlockSpec((TOPK, BLK), lambda i: (0, i)),
        ],
        out_shape=[
            jax.ShapeDtypeStruct((TOPK, n_tok), jnp.float32),
            jax.ShapeDtypeStruct((TOPK, n_tok), jnp.int32),
        ],
    )(x, W)
    return wt.T.astype(x.dtype), it.T
